# baseline (speedup 1.0000x reference)
import numpy as np
import jax
import jax.numpy as jnp
from jax.experimental import pallas as pl
from jax.experimental.pallas import tpu as pltpu

_HI = jax.lax.Precision.HIGHEST


def _make_body(c, hh, ww):
    inv_hw = np.float32(1.0 / (hh * ww))

    def body(xa_ref, xb_ref, ya_ref, yb_ref, yc_ref, yd_ref,
             wf_ref, w1_ref, w2_ref, o_ref):
        x = jnp.concatenate([xa_ref[0], xb_ref[0]], axis=0)

        def taps(y_ref):
            y6 = y_ref[0].reshape(hh // 4, 2, ww, 2, c // 128, 128)
            return (y6[:, 0, :, 0] + y6[:, 0, :, 1] +
                    y6[:, 1, :, 0] + y6[:, 1, :, 1])

        yp = (jnp.concatenate(
            [taps(ya_ref), taps(yb_ref), taps(yc_ref), taps(yd_ref)],
            axis=0) * np.float32(0.25)).reshape(hh * ww, c)

        ones = jnp.full((1, hh * ww), inv_hw, jnp.float32)
        xg = jnp.dot(ones, x, precision=_HI,
                     preferred_element_type=jnp.float32)
        yg = jnp.dot(ones, yp, precision=_HI,
                     preferred_element_type=jnp.float32)

        feat = jnp.concatenate([xg, yg], axis=1)
        common = jnp.maximum(
            jax.lax.dot_general(feat, wf_ref[...],
                                (((1,), (1,)), ((), ())), precision=_HI,
                                preferred_element_type=jnp.float32),
            0.0)
        xw = jax.nn.sigmoid(
            jnp.dot(common, w1_ref[...], precision=_HI,
                    preferred_element_type=jnp.float32))
        yw = jax.nn.sigmoid(
            jnp.dot(common, w2_ref[...], precision=_HI,
                    preferred_element_type=jnp.float32))

        o_ref[0] = x * xw + yw * yp

    return body


@jax.jit
def kernel(x, y, w_fc, w_fc1, w_fc2):
    B, C, H, W = x.shape
    assert y.shape == (B, C, 2 * H, 2 * W)
    hidden = w_fc.shape[1]

    xt = jax.lax.transpose(x.astype(jnp.float32), (0, 2, 3, 1))
    yt = jax.lax.transpose(y.astype(jnp.float32), (0, 2, 3, 1))
    xr = xt.reshape(B, H * W, C)
    yr = yt.reshape(B, 4 * H * W, C)

    out = pl.pallas_call(
        _make_body(C, H, W),
        grid=(B,),
        in_specs=[
            pl.BlockSpec((1, H * W // 2, C), lambda b: (b, 0, 0)),
            pl.BlockSpec((1, H * W // 2, C), lambda b: (b, 1, 0)),
            pl.BlockSpec((1, H * W, C), lambda b: (b, 0, 0)),
            pl.BlockSpec((1, H * W, C), lambda b: (b, 1, 0)),
            pl.BlockSpec((1, H * W, C), lambda b: (b, 2, 0)),
            pl.BlockSpec((1, H * W, C), lambda b: (b, 3, 0)),
            pl.BlockSpec((hidden, 2 * C), lambda b: (0, 0)),
            pl.BlockSpec((hidden, C), lambda b: (0, 0)),
            pl.BlockSpec((hidden, C), lambda b: (0, 0)),
        ],
        out_specs=pl.BlockSpec((1, H * W, C), lambda b: (b, 0, 0)),
        out_shape=jax.ShapeDtypeStruct((B, H * W, C), jnp.float32),
        compiler_params=pltpu.CompilerParams(
            dimension_semantics=("parallel",),
            vmem_limit_bytes=48 * 1024 * 1024),
    )(xr, xr,
      yr, yr, yr, yr,
      jax.lax.transpose(w_fc.astype(jnp.float32), (1, 0)),
      w_fc1.astype(jnp.float32), w_fc2.astype(jnp.float32))

    return jax.lax.transpose(out.reshape(B, H, W, C), (0, 3, 1, 2))
